# baseline (speedup 1.0000x reference)
import functools

import jax
import jax.numpy as jnp
from jax import lax
from jax.experimental import pallas as pl
from jax.experimental.pallas import tpu as pltpu
from jax.experimental.pallas import tpu_sc as plsc

N_NODES = 10000
N_EDGES = 320000
NPAD = 10240
D1 = 128
D2 = 16
NC = 2
NS = 16
NW = NC * NS
B = 80
NBLK = N_EDGES // (NW * B)
RING = 2
ROWS_PER_TILE = NPAD // NS

_MESH = plsc.VectorSubcoreMesh(
    core_axis_name="c", subcore_axis_name="s", num_cores=NC, num_subcores=NS)


def _fill_f32(ref, n, val):
    def body(i, _):
        ref[pl.ds(i * 16, 16)] = jnp.full((16,), val, jnp.float32)
        return 0
    lax.fori_loop(0, n // 16, body, 0)


def _deg_body(dst3d, degp0, degp1, deg_sh, idxbuf, onesbuf, zbuf, ssem):
    c = lax.axis_index("c")
    s = lax.axis_index("s")
    w = c * NS + s
    pltpu.sync_copy(dst3d.at[w], idxbuf)
    _fill_f32(zbuf, ROWS_PER_TILE, 0.0)
    _fill_f32(onesbuf, B, 1.0)
    pltpu.sync_copy(zbuf, deg_sh.at[pl.ds(s * ROWS_PER_TILE, ROWS_PER_TILE)])
    plsc.subcore_barrier()
    def fire(j, _):
        pltpu.async_copy(onesbuf, deg_sh.at[idxbuf.at[j]], ssem, add=True)
        return 0
    lax.fori_loop(0, NBLK, fire, 0)
    def drain(j, _):
        pltpu.make_async_copy(onesbuf, deg_sh.at[idxbuf.at[j]], ssem).wait()
        return 0
    lax.fori_loop(0, NBLK, drain, 0)
    plsc.subcore_barrier()
    sl = pl.ds(s * ROWS_PER_TILE, ROWS_PER_TILE)
    @pl.when(c == 0)
    def _():
        pltpu.sync_copy(deg_sh.at[sl], degp0.at[sl])
    @pl.when(c == 1)
    def _():
        pltpu.sync_copy(deg_sh.at[sl], degp1.at[sl])


_deg_kernel = functools.partial(
    pl.kernel, _deg_body,
    out_type=[jax.ShapeDtypeStruct((NPAD,), jnp.float32),
              jax.ShapeDtypeStruct((NPAD,), jnp.float32)],
    mesh=_MESH,
    scratch_types=[
        pltpu.VMEM_SHARED((NPAD,), jnp.float32),
        pltpu.VMEM((NBLK, B), jnp.int32),
        pltpu.VMEM((B,), jnp.float32),
        pltpu.VMEM((ROWS_PER_TILE,), jnp.float32),
        pltpu.SemaphoreType.DMA,
    ],
)()


B3 = 50
NBLK3 = N_EDGES // (NW * B3)
CHUNK = 25
NCHUNK = NBLK3 // CHUNK
RING3 = 3


def _agg_body(d, table, src3d, dst3d, accp, acc_sh, sbuf, dbuf, rows, gsem,
              isem):
    c = lax.axis_index("c")
    s = lax.axis_index("s")
    w = c * NS + s
    r0 = s * ROWS_PER_TILE

    def idx_load(ci, slot):
        return (
            pltpu.make_async_copy(src3d.at[w, ci], sbuf.at[slot], isem),
            pltpu.make_async_copy(dst3d.at[w, ci], dbuf.at[slot], isem),
        )

    for cp in idx_load(0, 0):
        cp.start()
    pltpu.sync_copy(table.at[pl.ds(r0, ROWS_PER_TILE)],
                    acc_sh.at[pl.ds(r0, ROWS_PER_TILE)])
    plsc.subcore_barrier()

    def chunk_body(ci, _):
        slot = lax.rem(ci, 2)
        for cp in idx_load(ci, slot):
            cp.wait()
        @pl.when(ci < NCHUNK - 1)
        def _():
            for cp in idx_load(ci + 1, 1 - slot):
                cp.start()

        def gather(j, r):
            return pltpu.make_async_copy(
                table.at[sbuf.at[slot, j]], rows.at[r], gsem.at[r])

        for r in range(RING3 - 1):
            gather(r, r).start()

        def blk(j, _):
            r = lax.rem(j, RING3)
            gather(j, r).wait()
            pltpu.sync_copy(rows.at[r], acc_sh.at[dbuf.at[slot, j]],
                            add=True)
            @pl.when(j + RING3 - 1 < CHUNK)
            def _():
                gather(j + RING3 - 1, lax.rem(j + RING3 - 1, RING3)).start()
            return 0
        lax.fori_loop(0, CHUNK, blk, 0)
        return 0
    lax.fori_loop(0, NCHUNK, chunk_body, 0)
    plsc.subcore_barrier()
    pltpu.sync_copy(acc_sh.at[pl.ds(r0, ROWS_PER_TILE)],
                    accp.at[c, pl.ds(r0, ROWS_PER_TILE)])


GRP = 5


def _agg16_body(table, src3d, dst3d, accp, acc_sh, sbuf, dbuf, rows, gsem,
                ssem):
    c = lax.axis_index("c")
    s = lax.axis_index("s")
    w = c * NS + s
    r0 = s * ROWS_PER_TILE
    pltpu.sync_copy(src3d.at[w], sbuf)
    pltpu.sync_copy(dst3d.at[w], dbuf)
    pltpu.sync_copy(table.at[pl.ds(r0, ROWS_PER_TILE)],
                    acc_sh.at[pl.ds(r0, ROWS_PER_TILE)])
    plsc.subcore_barrier()

    def gather(j, slot):
        return pltpu.make_async_copy(
            table.at[sbuf.at[j]], rows.at[slot], gsem.at[slot])

    def scat(j, slot):
        return pltpu.make_async_copy(
            rows.at[slot], acc_sh.at[dbuf.at[j]], ssem.at[slot])

    for r in range(GRP):
        gather(r, r).start()

    ngrp = NBLK // GRP
    def grp_body(g, _):
        base = lax.rem(g, 2) * GRP
        for r in range(GRP):
            j = g * GRP + r
            gather(j, base + r).wait()
            pltpu.async_copy(rows.at[base + r], acc_sh.at[dbuf.at[j]],
                             ssem.at[base + r], add=True)
        @pl.when(g < ngrp - 1)
        def _():
            base2 = lax.rem(g + 1, 2) * GRP
            for r in range(GRP):
                j2 = (g + 1) * GRP + r
                @pl.when(g >= 1)
                def _():
                    scat(j2 - 2 * GRP, base2 + r).wait()
                gather(j2, base2 + r).start()
        return 0
    lax.fori_loop(0, ngrp, grp_body, 0)
    for r in range(GRP):
        scat((ngrp - 2) * GRP + r, ((ngrp - 2) % 2) * GRP + r).wait()
        scat((ngrp - 1) * GRP + r, ((ngrp - 1) % 2) * GRP + r).wait()
    plsc.subcore_barrier()
    pltpu.sync_copy(acc_sh.at[pl.ds(r0, ROWS_PER_TILE)],
                    accp.at[c, pl.ds(r0, ROWS_PER_TILE)])


_agg16_kernel = functools.partial(
    pl.kernel, _agg16_body,
    out_type=jax.ShapeDtypeStruct((NC, NPAD, D2), jnp.float32),
    mesh=_MESH,
    compiler_params=pltpu.CompilerParams(use_tc_tiling_on_sc=False),
    scratch_types=[
        pltpu.VMEM_SHARED((NPAD, D2), jnp.float32),
        pltpu.VMEM((NBLK, B), jnp.int32),
        pltpu.VMEM((NBLK, B), jnp.int32),
        pltpu.VMEM((2 * GRP, B, D2), jnp.float32),
        pltpu.SemaphoreType.DMA((2 * GRP,)),
        pltpu.SemaphoreType.DMA((2 * GRP,)),
    ],
)()


def _agg_kernel(d):
    return functools.partial(
        pl.kernel, functools.partial(_agg_body, d),
        out_type=jax.ShapeDtypeStruct((NC, NPAD, d), jnp.float32),
        mesh=_MESH,
        compiler_params=pltpu.CompilerParams(use_tc_tiling_on_sc=(d == D1)),
        scratch_types=[
            pltpu.VMEM_SHARED((NPAD, d), jnp.float32),
            pltpu.VMEM((2, CHUNK, B3), jnp.int32),
            pltpu.VMEM((2, CHUNK, B3), jnp.int32),
            pltpu.VMEM((RING3, B3, d), jnp.float32),
            pltpu.SemaphoreType.DMA((RING3,)),
            pltpu.SemaphoreType.DMA,
        ],
    )()


def _scale_body(x_ref, degp0_ref, degp1_ref, xs_ref, dinv_ref):
    deg = degp0_ref[...] + degp1_ref[...] + 1.0
    dinv = lax.rsqrt(deg)
    xs_ref[...] = x_ref[...] * dinv
    dinv_ref[...] = dinv


def _mm_body(accp_ref, xs_ref, dinv_ref, W1_ref, b1_ref, W2_ref, b2_ref,
             ys_ref):
    dinv = dinv_ref[...]
    t1 = dinv * (accp_ref[0] + accp_ref[1] - xs_ref[...])
    h1 = jnp.dot(t1, W1_ref[...], preferred_element_type=jnp.float32) \
        + b1_ref[...]
    z = jnp.where(h1 > 0, h1, jnp.exp(h1) - 1.0)
    y = jnp.dot(z, W2_ref[...], preferred_element_type=jnp.float32)
    ys_ref[...] = dinv * y


def _pool_body(acc2p_ref, ys_ref, dinv_ref, batch_ref, b2_ref, out_ref):
    t2 = dinv_ref[...] * (acc2p_ref[0] + acc2p_ref[1] - ys_ref[...]) \
        + b2_ref[...]
    b = jnp.broadcast_to(batch_ref[...], (D2, NPAD))
    gid = lax.broadcasted_iota(jnp.int32, (D2, NPAD), 0)
    m = (b == gid).astype(jnp.float32)
    cnt = jnp.sum(m, axis=1, keepdims=True)
    pool = jnp.dot(m, t2, preferred_element_type=jnp.float32)
    out_ref[...] = pool / jnp.maximum(cnt, 1.0)


def kernel(x, edge_idx, batch, W1, b1, W2, b2):
    src3d = edge_idx[0].reshape(NW, NCHUNK, CHUNK, B3)
    dst3d = edge_idx[1].reshape(NW, NCHUNK, CHUNK, B3)
    dst3d_l1 = edge_idx[1].reshape(NW, NBLK, B)
    pad = NPAD - N_NODES
    x_p = jnp.concatenate([x, jnp.zeros((pad, D1), jnp.float32)])
    batch_p = jnp.concatenate(
        [batch, jnp.full((pad,), D2, jnp.int32)]).reshape(1, NPAD)

    degp0, degp1 = _deg_kernel(dst3d_l1)
    degp0 = degp0.reshape(NPAD, 1)
    degp1 = degp1.reshape(NPAD, 1)
    xs, dinv = pl.pallas_call(
        _scale_body,
        out_shape=[jax.ShapeDtypeStruct((NPAD, D1), jnp.float32),
                   jax.ShapeDtypeStruct((NPAD, 1), jnp.float32)],
    )(x_p, degp0, degp1)
    accp = _agg_kernel(D1)(xs, src3d, dst3d)
    ys = pl.pallas_call(
        _mm_body,
        out_shape=jax.ShapeDtypeStruct((NPAD, D2), jnp.float32),
    )(accp, xs, dinv, W1, b1.reshape(1, D1), W2, b2.reshape(1, D2))
    src3d_l1 = edge_idx[0].reshape(NW, NBLK, B)
    acc2p = _agg16_kernel(ys, src3d_l1, dst3d_l1)
    out = pl.pallas_call(
        _pool_body,
        out_shape=jax.ShapeDtypeStruct((D2, D2), jnp.float32),
    )(acc2p, ys, dinv, batch_p, b2.reshape(1, D2))
    return out


# speedup vs baseline: 1.1447x; 1.1447x over previous
import functools

import jax
import jax.numpy as jnp
from jax import lax
from jax.experimental import pallas as pl
from jax.experimental.pallas import tpu as pltpu
from jax.experimental.pallas import tpu_sc as plsc

N_NODES = 10000
N_EDGES = 320000
NPAD = 10240
D1 = 128
D2 = 16
NC = 2
NS = 16
NW = NC * NS
B = 80
NBLK = N_EDGES // (NW * B)
RING = 2
ROWS_PER_TILE = NPAD // NS

_MESH = plsc.VectorSubcoreMesh(
    core_axis_name="c", subcore_axis_name="s", num_cores=NC, num_subcores=NS)


def _fill_f32(ref, n, val):
    def body(i, _):
        ref[pl.ds(i * 16, 16)] = jnp.full((16,), val, jnp.float32)
        return 0
    lax.fori_loop(0, n // 16, body, 0)


def _deg_body(dst3d, degp0, degp1, deg_sh, idxbuf, onesbuf, zbuf, ssem):
    c = lax.axis_index("c")
    s = lax.axis_index("s")
    w = c * NS + s
    pltpu.sync_copy(dst3d.at[w], idxbuf)
    _fill_f32(zbuf, ROWS_PER_TILE, 0.0)
    _fill_f32(onesbuf, B, 1.0)
    pltpu.sync_copy(zbuf, deg_sh.at[pl.ds(s * ROWS_PER_TILE, ROWS_PER_TILE)])
    plsc.subcore_barrier()
    def fire(j, _):
        pltpu.async_copy(onesbuf, deg_sh.at[idxbuf.at[j]], ssem, add=True)
        return 0
    lax.fori_loop(0, NBLK, fire, 0)
    def drain(j, _):
        pltpu.make_async_copy(onesbuf, deg_sh.at[idxbuf.at[j]], ssem).wait()
        return 0
    lax.fori_loop(0, NBLK, drain, 0)
    plsc.subcore_barrier()
    sl = pl.ds(s * ROWS_PER_TILE, ROWS_PER_TILE)
    @pl.when(c == 0)
    def _():
        pltpu.sync_copy(deg_sh.at[sl], degp0.at[sl])
    @pl.when(c == 1)
    def _():
        pltpu.sync_copy(deg_sh.at[sl], degp1.at[sl])


_deg_kernel = functools.partial(
    pl.kernel, _deg_body,
    out_type=[jax.ShapeDtypeStruct((NPAD,), jnp.float32),
              jax.ShapeDtypeStruct((NPAD,), jnp.float32)],
    mesh=_MESH,
    scratch_types=[
        pltpu.VMEM_SHARED((NPAD,), jnp.float32),
        pltpu.VMEM((NBLK, B), jnp.int32),
        pltpu.VMEM((B,), jnp.float32),
        pltpu.VMEM((ROWS_PER_TILE,), jnp.float32),
        pltpu.SemaphoreType.DMA,
    ],
)()


B3 = 100
NBLK3 = N_EDGES // (NW * B3)
CHUNK = 20
NCHUNK = NBLK3 // CHUNK
RING3 = 2


def _agg_body(d, table, src3d, dst3d, accp, acc_sh, sbuf, dbuf, rows, gsem,
              isem):
    c = lax.axis_index("c")
    s = lax.axis_index("s")
    w = c * NS + s
    r0 = s * ROWS_PER_TILE

    def idx_load(ci, slot):
        return (
            pltpu.make_async_copy(src3d.at[w, ci], sbuf.at[slot], isem),
            pltpu.make_async_copy(dst3d.at[w, ci], dbuf.at[slot], isem),
        )

    for cp in idx_load(0, 0):
        cp.start()
    pltpu.sync_copy(table.at[pl.ds(r0, ROWS_PER_TILE)],
                    acc_sh.at[pl.ds(r0, ROWS_PER_TILE)])
    plsc.subcore_barrier()

    def chunk_body(ci, _):
        slot = lax.rem(ci, 2)
        for cp in idx_load(ci, slot):
            cp.wait()
        @pl.when(ci < NCHUNK - 1)
        def _():
            for cp in idx_load(ci + 1, 1 - slot):
                cp.start()

        def gather(j, r):
            return pltpu.make_async_copy(
                table.at[sbuf.at[slot, j]], rows.at[r], gsem.at[r])

        for r in range(RING3):
            gather(r, r).start()

        def blk(j, _):
            r = lax.rem(j, RING3)
            gather(j, r).wait()
            pltpu.sync_copy(rows.at[r], acc_sh.at[dbuf.at[slot, j]],
                            add=True)
            @pl.when(j + RING3 < CHUNK)
            def _():
                gather(j + RING3, r).start()
            return 0
        lax.fori_loop(0, CHUNK, blk, 0)
        return 0
    lax.fori_loop(0, NCHUNK, chunk_body, 0)
    plsc.subcore_barrier()
    pltpu.sync_copy(acc_sh.at[pl.ds(r0, ROWS_PER_TILE)],
                    accp.at[c, pl.ds(r0, ROWS_PER_TILE)])


GRP = 5


def _agg16_body(table, src3d, dst3d, accp, acc_sh, sbuf, dbuf, rows, gsem,
                ssem):
    c = lax.axis_index("c")
    s = lax.axis_index("s")
    w = c * NS + s
    r0 = s * ROWS_PER_TILE
    pltpu.sync_copy(src3d.at[w], sbuf)
    pltpu.sync_copy(dst3d.at[w], dbuf)
    pltpu.sync_copy(table.at[pl.ds(r0, ROWS_PER_TILE)],
                    acc_sh.at[pl.ds(r0, ROWS_PER_TILE)])
    plsc.subcore_barrier()

    def gather(j, slot):
        return pltpu.make_async_copy(
            table.at[sbuf.at[j]], rows.at[slot], gsem.at[slot])

    def scat(j, slot):
        return pltpu.make_async_copy(
            rows.at[slot], acc_sh.at[dbuf.at[j]], ssem.at[slot])

    for r in range(GRP):
        gather(r, r).start()

    ngrp = NBLK // GRP
    def grp_body(g, _):
        base = lax.rem(g, 2) * GRP
        for r in range(GRP):
            j = g * GRP + r
            gather(j, base + r).wait()
            pltpu.async_copy(rows.at[base + r], acc_sh.at[dbuf.at[j]],
                             ssem.at[base + r], add=True)
        @pl.when(g < ngrp - 1)
        def _():
            base2 = lax.rem(g + 1, 2) * GRP
            for r in range(GRP):
                j2 = (g + 1) * GRP + r
                @pl.when(g >= 1)
                def _():
                    scat(j2 - 2 * GRP, base2 + r).wait()
                gather(j2, base2 + r).start()
        return 0
    lax.fori_loop(0, ngrp, grp_body, 0)
    for r in range(GRP):
        scat((ngrp - 2) * GRP + r, ((ngrp - 2) % 2) * GRP + r).wait()
        scat((ngrp - 1) * GRP + r, ((ngrp - 1) % 2) * GRP + r).wait()
    plsc.subcore_barrier()
    pltpu.sync_copy(acc_sh.at[pl.ds(r0, ROWS_PER_TILE)],
                    accp.at[c, pl.ds(r0, ROWS_PER_TILE)])


_agg16_kernel = functools.partial(
    pl.kernel, _agg16_body,
    out_type=jax.ShapeDtypeStruct((NC, NPAD, D2), jnp.float32),
    mesh=_MESH,
    compiler_params=pltpu.CompilerParams(use_tc_tiling_on_sc=False),
    scratch_types=[
        pltpu.VMEM_SHARED((NPAD, D2), jnp.float32),
        pltpu.VMEM((NBLK, B), jnp.int32),
        pltpu.VMEM((NBLK, B), jnp.int32),
        pltpu.VMEM((2 * GRP, B, D2), jnp.float32),
        pltpu.SemaphoreType.DMA((2 * GRP,)),
        pltpu.SemaphoreType.DMA((2 * GRP,)),
    ],
)()


def _agg_kernel(d):
    return functools.partial(
        pl.kernel, functools.partial(_agg_body, d),
        out_type=jax.ShapeDtypeStruct((NC, NPAD, d), jnp.float32),
        mesh=_MESH,
        compiler_params=pltpu.CompilerParams(use_tc_tiling_on_sc=(d == D1)),
        scratch_types=[
            pltpu.VMEM_SHARED((NPAD, d), jnp.float32),
            pltpu.VMEM((2, CHUNK, B3), jnp.int32),
            pltpu.VMEM((2, CHUNK, B3), jnp.int32),
            pltpu.VMEM((RING3, B3, d), jnp.float32),
            pltpu.SemaphoreType.DMA((RING3,)),
            pltpu.SemaphoreType.DMA,
        ],
    )()


def _scale_body(x_ref, degp0_ref, degp1_ref, xs_ref, dinv_ref):
    deg = degp0_ref[...] + degp1_ref[...] + 1.0
    dinv = lax.rsqrt(deg)
    xs_ref[...] = x_ref[...] * dinv
    dinv_ref[...] = dinv


def _mm_body(accp_ref, xs_ref, dinv_ref, W1_ref, b1_ref, W2_ref, b2_ref,
             ys_ref):
    dinv = dinv_ref[...]
    t1 = dinv * (accp_ref[0] + accp_ref[1] - xs_ref[...])
    h1 = jnp.dot(t1, W1_ref[...], preferred_element_type=jnp.float32) \
        + b1_ref[...]
    z = jnp.where(h1 > 0, h1, jnp.exp(h1) - 1.0)
    y = jnp.dot(z, W2_ref[...], preferred_element_type=jnp.float32)
    ys_ref[...] = dinv * y


def _pool_body(acc2p_ref, ys_ref, dinv_ref, batch_ref, b2_ref, out_ref):
    t2 = dinv_ref[...] * (acc2p_ref[0] + acc2p_ref[1] - ys_ref[...]) \
        + b2_ref[...]
    b = jnp.broadcast_to(batch_ref[...], (D2, NPAD))
    gid = lax.broadcasted_iota(jnp.int32, (D2, NPAD), 0)
    m = (b == gid).astype(jnp.float32)
    cnt = jnp.sum(m, axis=1, keepdims=True)
    pool = jnp.dot(m, t2, preferred_element_type=jnp.float32)
    out_ref[...] = pool / jnp.maximum(cnt, 1.0)


def kernel(x, edge_idx, batch, W1, b1, W2, b2):
    src3d = edge_idx[0].reshape(NW, NCHUNK, CHUNK, B3)
    dst3d = edge_idx[1].reshape(NW, NCHUNK, CHUNK, B3)
    dst3d_l1 = edge_idx[1].reshape(NW, NBLK, B)
    pad = NPAD - N_NODES
    x_p = jnp.concatenate([x, jnp.zeros((pad, D1), jnp.float32)])
    batch_p = jnp.concatenate(
        [batch, jnp.full((pad,), D2, jnp.int32)]).reshape(1, NPAD)

    degp0, degp1 = _deg_kernel(dst3d_l1)
    degp0 = degp0.reshape(NPAD, 1)
    degp1 = degp1.reshape(NPAD, 1)
    xs, dinv = pl.pallas_call(
        _scale_body,
        out_shape=[jax.ShapeDtypeStruct((NPAD, D1), jnp.float32),
                   jax.ShapeDtypeStruct((NPAD, 1), jnp.float32)],
    )(x_p, degp0, degp1)
    accp = _agg_kernel(D1)(xs, src3d, dst3d)
    ys = pl.pallas_call(
        _mm_body,
        out_shape=jax.ShapeDtypeStruct((NPAD, D2), jnp.float32),
    )(accp, xs, dinv, W1, b1.reshape(1, D1), W2, b2.reshape(1, D2))
    src3d_l1 = edge_idx[0].reshape(NW, NBLK, B)
    acc2p = _agg16_kernel(ys, src3d_l1, dst3d_l1)
    out = pl.pallas_call(
        _pool_body,
        out_shape=jax.ShapeDtypeStruct((D2, D2), jnp.float32),
    )(acc2p, ys, dinv, batch_p, b2.reshape(1, D2))
    return out
